# Initial kernel scaffold; baseline (speedup 1.0000x reference)
import jax
import jax.numpy as jnp
from jax.experimental import pallas as pl


def kernel(src_vts, src_faces, trg_vts, trg_color):
    raise NotImplementedError("write your pallas kernel here")


# speedup vs baseline: 1.3690x; 1.3690x over previous
import functools

import jax
import jax.numpy as jnp
from jax.experimental import pallas as pl
from jax.experimental.pallas import tpu as pltpu

N = 4096
M = 4096
F = 16384
TILE = 256
NTILES = N // TILE
KPAD = 8


def _chamfer_grad_kernel(verts_ref, trgT_ref, trg_ref, t2_ref, out_ref, colmin_ref):
    phase = pl.program_id(0)
    tile = pl.program_id(1)
    vtile = verts_ref[...]
    g = jnp.dot(vtile, trgT_ref[...], preferred_element_type=jnp.float32)
    v2 = jnp.sum(vtile * vtile, axis=1, keepdims=True)
    d2 = jnp.maximum(v2 + t2_ref[...] - 2.0 * g, 0.0)

    @pl.when(phase == 0)
    def _():
        part = jnp.min(d2, axis=0, keepdims=True)

        @pl.when(tile == 0)
        def _():
            colmin_ref[...] = part

        @pl.when(tile > 0)
        def _():
            colmin_ref[...] = jnp.minimum(colmin_ref[...], part)

    @pl.when(phase == 1)
    def _():
        rowmin = jnp.min(d2, axis=1, keepdims=True)
        ohr = (d2 == rowmin).astype(jnp.float32)
        rs = jnp.sum(ohr, axis=1, keepdims=True)
        sel = jnp.dot(ohr, trg_ref[...], preferred_element_type=jnp.float32)
        ga = (2.0 / N) * (vtile - sel / rs)
        c = (d2 == colmin_ref[...]).astype(jnp.float32)
        cnt = jnp.sum(c, axis=1, keepdims=True)
        selc = jnp.dot(c, trg_ref[...], preferred_element_type=jnp.float32)
        gb = (2.0 / M) * (vtile * cnt - selc)
        out_ref[...] = ga + gb


@functools.partial(jax.jit, static_argnames=())
def _chamfer_grad(verts_p, trgT_p, trg_p, t2):
    return pl.pallas_call(
        _chamfer_grad_kernel,
        grid=(2, NTILES),
        in_specs=[
            pl.BlockSpec((TILE, KPAD), lambda p, t: (t, 0)),
            pl.BlockSpec((KPAD, M), lambda p, t: (0, 0)),
            pl.BlockSpec((M, KPAD), lambda p, t: (0, 0)),
            pl.BlockSpec((1, M), lambda p, t: (0, 0)),
        ],
        out_specs=pl.BlockSpec((TILE, KPAD), lambda p, t: (t, 0)),
        out_shape=jax.ShapeDtypeStruct((N, KPAD), jnp.float32),
        scratch_shapes=[pltpu.VMEM((1, M), jnp.float32)],
    )(verts_p, trgT_p, trg_p, t2)


def _mesh_grads_jax(verts, f0, f1, f2):
    v0 = verts[f0]
    v1 = verts[f1]
    v2 = verts[f2]
    d0 = v0 - v1
    d1 = v1 - v2
    d2 = v2 - v0
    e0 = jnp.sqrt(jnp.sum(d0 * d0, axis=1))
    e1 = jnp.sqrt(jnp.sum(d1 * d1, axis=1))
    e2 = jnp.sqrt(jnp.sum(d2 * d2, axis=1))
    c0 = (2.0 / F) * (2.0 * e0 - e1 - e2) / e0
    c1 = (2.0 / F) * (2.0 * e1 - e0 - e2) / e1
    c2 = (2.0 / F) * (2.0 * e2 - e0 - e1) / e2
    gv0 = c0[:, None] * d0 - c2[:, None] * d2
    gv1 = -c0[:, None] * d0 + c1[:, None] * d1
    gv2 = -c1[:, None] * d1 + c2[:, None] * d2
    g_edge = jnp.zeros_like(verts)
    g_edge = g_edge.at[f0].add(gv0)
    g_edge = g_edge.at[f1].add(gv1)
    g_edge = g_edge.at[f2].add(gv2)

    nbr = jnp.zeros_like(verts)
    nbr = nbr.at[f0].add(v1 + v2)
    nbr = nbr.at[f1].add(v0 + v2)
    nbr = nbr.at[f2].add(v0 + v1)
    deg = jnp.zeros((N,), jnp.float32)
    deg = deg.at[f0].add(2.0).at[f1].add(2.0).at[f2].add(2.0)
    degc = jnp.maximum(deg, 1.0)
    lap = nbr / degc[:, None] - verts
    nl = jnp.sqrt(jnp.sum(lap * lap, axis=1))
    u = lap / (N * nl[:, None])
    w = u / degc[:, None]
    wsum = jnp.zeros_like(verts)
    w0 = w[f0]
    w1 = w[f1]
    w2 = w[f2]
    wsum = wsum.at[f0].add(w1 + w2)
    wsum = wsum.at[f1].add(w0 + w2)
    wsum = wsum.at[f2].add(w0 + w1)
    g_lap = wsum - u
    return g_edge + 0.05 * g_lap


def kernel(src_vts, src_faces, trg_vts, trg_color):
    del trg_color
    lr, mom = 0.01, 0.99
    f0 = src_faces[:, 0]
    f1 = src_faces[:, 1]
    f2 = src_faces[:, 2]
    pad = jnp.zeros((N, KPAD - 3), jnp.float32)
    trg_p = jnp.concatenate([trg_vts, pad], axis=1)
    trgT_p = trg_p.T
    t2 = jnp.sum(trg_vts * trg_vts, axis=1)[None, :]

    deform = jnp.zeros_like(src_vts)
    vel = jnp.zeros_like(src_vts)
    for _ in range(3):
        verts = src_vts + deform
        verts_p = jnp.concatenate([verts, pad], axis=1)
        g_ch = _chamfer_grad(verts_p, trgT_p, trg_p, t2)[:, :3]
        g_mesh = _mesh_grads_jax(verts, f0, f1, f2)
        g = 0.8 * g_ch + g_mesh
        vel = mom * vel + g
        deform = deform - lr * vel
    return src_vts + deform


# speedup vs baseline: 17.8880x; 13.0667x over previous
import functools

import jax
import jax.numpy as jnp
from jax import lax
from jax.experimental import pallas as pl
from jax.experimental.pallas import tpu as pltpu
from jax.experimental.pallas import tpu_sc as plsc

N = 4096
M = 4096
F = 16384
TILE = 256
NTILES = N // TILE
KPAD = 8

NSUB = 16
FACES_PER_SUB = F // NSUB
NGROUPS = FACES_PER_SUB // 16
VPS = N // NSUB
A1_PER = 7 * 256
A1_TOT = NSUB * A1_PER
A2_PER = 3 * 256
A2_TOT = NSUB * A2_PER


def _chamfer_grad_kernel(verts_ref, trgT_ref, trg_ref, t2_ref, out_ref, colmin_ref):
    phase = pl.program_id(0)
    tile = pl.program_id(1)
    vtile = verts_ref[...]
    g = jnp.dot(vtile, trgT_ref[...], preferred_element_type=jnp.float32)
    v2 = jnp.sum(vtile * vtile, axis=1, keepdims=True)
    d2 = jnp.maximum(v2 + t2_ref[...] - 2.0 * g, 0.0)

    @pl.when(phase == 0)
    def _():
        part = jnp.min(d2, axis=0, keepdims=True)

        @pl.when(tile == 0)
        def _():
            colmin_ref[...] = part

        @pl.when(tile > 0)
        def _():
            colmin_ref[...] = jnp.minimum(colmin_ref[...], part)

    @pl.when(phase == 1)
    def _():
        rowmin = jnp.min(d2, axis=1, keepdims=True)
        ohr = (d2 == rowmin).astype(jnp.float32)
        rs = jnp.sum(ohr, axis=1, keepdims=True)
        sel = jnp.dot(ohr, trg_ref[...], preferred_element_type=jnp.float32)
        ga = (2.0 / N) * (vtile - sel / rs)
        c = (d2 == colmin_ref[...]).astype(jnp.float32)
        cnt = jnp.sum(c, axis=1, keepdims=True)
        selc = jnp.dot(c, trg_ref[...], preferred_element_type=jnp.float32)
        gb = (2.0 / M) * (vtile * cnt - selc)
        out_ref[...] = ga + gb


def _chamfer_grad(verts_p, trgT_p, trg_p, t2):
    return pl.pallas_call(
        _chamfer_grad_kernel,
        grid=(2, NTILES),
        in_specs=[
            pl.BlockSpec((TILE, KPAD), lambda p, t: (t, 0)),
            pl.BlockSpec((KPAD, M), lambda p, t: (0, 0)),
            pl.BlockSpec((M, KPAD), lambda p, t: (0, 0)),
            pl.BlockSpec((1, M), lambda p, t: (0, 0)),
        ],
        out_specs=pl.BlockSpec((TILE, KPAD), lambda p, t: (t, 0)),
        out_shape=jax.ShapeDtypeStruct((N, KPAD), jnp.float32),
        scratch_shapes=[pltpu.VMEM((1, M), jnp.float32)],
    )(verts_p, trgT_p, trg_p, t2)


def _sqrt16(n):
    i = plsc.bitcast(n, jnp.int32)
    y = plsc.bitcast(jnp.int32(0x5F3759DF) - lax.shift_right_logical(i, 1),
                     jnp.float32)
    half = 0.5 * n
    y = y * (1.5 - half * y * y)
    y = y * (1.5 - half * y * y)
    y = y * (1.5 - half * y * y)
    return n * y


def _zero_fill(ref, nrows):
    z = jnp.zeros((16,), jnp.float32)

    def body(r, carry):
        ref[pl.ds(r * 16, 16)] = z
        return carry

    lax.fori_loop(0, nrows, body, 0, unroll=8)


def _mesh_sc_body(vh, fh, out,
                  lvx, lvy, lvz, lf0, lf1, lf2,
                  acc1, red1, acc2, red2, rtmp1, rtmp2,
                  lux, luy, luz, lwx, lwy, lwz,
                  fwx, fwy, fwz, lg,
                  shared1, sharedw, shared2):
    cid = lax.axis_index("c")
    s = lax.axis_index("s")

    pltpu.sync_copy(vh.at[pl.ds(0, N)], lvx)
    pltpu.sync_copy(vh.at[pl.ds(N, N)], lvy)
    pltpu.sync_copy(vh.at[pl.ds(2 * N, N)], lvz)
    pltpu.sync_copy(fh.at[pl.ds(s * FACES_PER_SUB, FACES_PER_SUB)], lf0)
    pltpu.sync_copy(fh.at[pl.ds(F + s * FACES_PER_SUB, FACES_PER_SUB)], lf1)
    pltpu.sync_copy(fh.at[pl.ds(2 * F + s * FACES_PER_SUB, FACES_PER_SUB)], lf2)
    _zero_fill(acc1, A1_PER * NSUB // 16)

    two_over_f = jnp.float32(2.0 / F)

    def face1(gi, carry):
        base = gi * 16
        ia = lf0[pl.ds(base, 16)]
        ib = lf1[pl.ds(base, 16)]
        ic = lf2[pl.ds(base, 16)]
        vax = plsc.load_gather(lvx, [ia])
        vay = plsc.load_gather(lvy, [ia])
        vaz = plsc.load_gather(lvz, [ia])
        vbx = plsc.load_gather(lvx, [ib])
        vby = plsc.load_gather(lvy, [ib])
        vbz = plsc.load_gather(lvz, [ib])
        vcx = plsc.load_gather(lvx, [ic])
        vcy = plsc.load_gather(lvy, [ic])
        vcz = plsc.load_gather(lvz, [ic])
        d0x, d0y, d0z = vax - vbx, vay - vby, vaz - vbz
        d1x, d1y, d1z = vbx - vcx, vby - vcy, vbz - vcz
        d2x, d2y, d2z = vcx - vax, vcy - vay, vcz - vaz
        e0 = _sqrt16(d0x * d0x + d0y * d0y + d0z * d0z)
        e1 = _sqrt16(d1x * d1x + d1y * d1y + d1z * d1z)
        e2 = _sqrt16(d2x * d2x + d2y * d2y + d2z * d2z)
        c0 = two_over_f * (2.0 * e0 - e1 - e2) / e0
        c1 = two_over_f * (2.0 * e1 - e0 - e2) / e1
        c2 = two_over_f * (2.0 * e2 - e0 - e1) / e2
        gv0x, gv0y, gv0z = c0 * d0x - c2 * d2x, c0 * d0y - c2 * d2y, c0 * d0z - c2 * d2z
        gv1x, gv1y, gv1z = c1 * d1x - c0 * d0x, c1 * d1y - c0 * d0y, c1 * d1z - c0 * d0z
        gv2x, gv2y, gv2z = c2 * d2x - c1 * d1x, c2 * d2y - c1 * d1y, c2 * d2z - c1 * d1z
        ja = lax.shift_right_logical(ia, 8) * A1_PER + jnp.bitwise_and(ia, 255)
        jb = lax.shift_right_logical(ib, 8) * A1_PER + jnp.bitwise_and(ib, 255)
        jc = lax.shift_right_logical(ic, 8) * A1_PER + jnp.bitwise_and(ic, 255)
        plsc.addupdate_scatter(acc1, [ja], vbx + vcx)
        plsc.addupdate_scatter(acc1, [ja + 256], vby + vcy)
        plsc.addupdate_scatter(acc1, [ja + 512], vbz + vcz)
        plsc.addupdate_scatter(acc1, [jb], vax + vcx)
        plsc.addupdate_scatter(acc1, [jb + 256], vay + vcy)
        plsc.addupdate_scatter(acc1, [jb + 512], vaz + vcz)
        plsc.addupdate_scatter(acc1, [jc], vax + vbx)
        plsc.addupdate_scatter(acc1, [jc + 256], vay + vby)
        plsc.addupdate_scatter(acc1, [jc + 512], vaz + vbz)
        twos = jnp.full((16,), 2.0, jnp.float32)
        plsc.addupdate_scatter(acc1, [ja + 768], twos)
        plsc.addupdate_scatter(acc1, [jb + 768], twos)
        plsc.addupdate_scatter(acc1, [jc + 768], twos)
        plsc.addupdate_scatter(acc1, [ja + 1024], gv0x)
        plsc.addupdate_scatter(acc1, [ja + 1280], gv0y)
        plsc.addupdate_scatter(acc1, [ja + 1536], gv0z)
        plsc.addupdate_scatter(acc1, [jb + 1024], gv1x)
        plsc.addupdate_scatter(acc1, [jb + 1280], gv1y)
        plsc.addupdate_scatter(acc1, [jb + 1536], gv1z)
        plsc.addupdate_scatter(acc1, [jc + 1024], gv2x)
        plsc.addupdate_scatter(acc1, [jc + 1280], gv2y)
        plsc.addupdate_scatter(acc1, [jc + 1536], gv2z)
        return carry

    lax.fori_loop(0, NGROUPS, face1, 0)

    pltpu.sync_copy(acc1, shared1.at[pl.ds(s * A1_TOT, A1_TOT)])
    plsc.subcore_barrier()
    _zero_fill(red1, A1_PER // 16)

    def red1_body(t, carry):
        pltpu.sync_copy(shared1.at[pl.ds(t * A1_TOT + s * A1_PER, A1_PER)], rtmp1)

        def addrow(r, c2_):
            red1[pl.ds(r * 16, 16)] = red1[pl.ds(r * 16, 16)] + rtmp1[pl.ds(r * 16, 16)]
            return c2_

        lax.fori_loop(0, A1_PER // 16, addrow, 0, unroll=8)
        return carry

    lax.fori_loop(0, NSUB, red1_body, 0)

    inv_n = jnp.float32(1.0 / N)

    def p3(r, carry):
        nbx = red1[pl.ds(r * 16, 16)]
        nby = red1[pl.ds(256 + r * 16, 16)]
        nbz = red1[pl.ds(512 + r * 16, 16)]
        dg = red1[pl.ds(768 + r * 16, 16)]
        degc = jnp.maximum(dg, 1.0)
        vx = lvx[pl.ds(s * VPS + r * 16, 16)]
        vy = lvy[pl.ds(s * VPS + r * 16, 16)]
        vz = lvz[pl.ds(s * VPS + r * 16, 16)]
        lapx = nbx / degc - vx
        lapy = nby / degc - vy
        lapz = nbz / degc - vz
        nl = _sqrt16(lapx * lapx + lapy * lapy + lapz * lapz)
        ux = inv_n * (lapx / nl)
        uy = inv_n * (lapy / nl)
        uz = inv_n * (lapz / nl)
        lux[pl.ds(r * 16, 16)] = ux
        luy[pl.ds(r * 16, 16)] = uy
        luz[pl.ds(r * 16, 16)] = uz
        lwx[pl.ds(r * 16, 16)] = ux / degc
        lwy[pl.ds(r * 16, 16)] = uy / degc
        lwz[pl.ds(r * 16, 16)] = uz / degc
        return carry

    lax.fori_loop(0, VPS // 16, p3, 0)

    pltpu.sync_copy(lwx, sharedw.at[pl.ds(s * VPS, VPS)])
    pltpu.sync_copy(lwy, sharedw.at[pl.ds(N + s * VPS, VPS)])
    pltpu.sync_copy(lwz, sharedw.at[pl.ds(2 * N + s * VPS, VPS)])
    plsc.subcore_barrier()
    pltpu.sync_copy(sharedw.at[pl.ds(0, N)], fwx)
    pltpu.sync_copy(sharedw.at[pl.ds(N, N)], fwy)
    pltpu.sync_copy(sharedw.at[pl.ds(2 * N, N)], fwz)
    _zero_fill(acc2, A2_PER * NSUB // 16)

    def face2(gi, carry):
        base = gi * 16
        ia = lf0[pl.ds(base, 16)]
        ib = lf1[pl.ds(base, 16)]
        ic = lf2[pl.ds(base, 16)]
        wax = plsc.load_gather(fwx, [ia])
        way = plsc.load_gather(fwy, [ia])
        waz = plsc.load_gather(fwz, [ia])
        wbx = plsc.load_gather(fwx, [ib])
        wby = plsc.load_gather(fwy, [ib])
        wbz = plsc.load_gather(fwz, [ib])
        wcx = plsc.load_gather(fwx, [ic])
        wcy = plsc.load_gather(fwy, [ic])
        wcz = plsc.load_gather(fwz, [ic])
        ja = lax.shift_right_logical(ia, 8) * A2_PER + jnp.bitwise_and(ia, 255)
        jb = lax.shift_right_logical(ib, 8) * A2_PER + jnp.bitwise_and(ib, 255)
        jc = lax.shift_right_logical(ic, 8) * A2_PER + jnp.bitwise_and(ic, 255)
        plsc.addupdate_scatter(acc2, [ja], wbx + wcx)
        plsc.addupdate_scatter(acc2, [ja + 256], wby + wcy)
        plsc.addupdate_scatter(acc2, [ja + 512], wbz + wcz)
        plsc.addupdate_scatter(acc2, [jb], wax + wcx)
        plsc.addupdate_scatter(acc2, [jb + 256], way + wcy)
        plsc.addupdate_scatter(acc2, [jb + 512], waz + wcz)
        plsc.addupdate_scatter(acc2, [jc], wax + wbx)
        plsc.addupdate_scatter(acc2, [jc + 256], way + wby)
        plsc.addupdate_scatter(acc2, [jc + 512], waz + wbz)
        return carry

    lax.fori_loop(0, NGROUPS, face2, 0)

    pltpu.sync_copy(acc2, shared2.at[pl.ds(s * A2_TOT, A2_TOT)])
    plsc.subcore_barrier()
    _zero_fill(red2, A2_PER // 16)

    def red2_body(t, carry):
        pltpu.sync_copy(shared2.at[pl.ds(t * A2_TOT + s * A2_PER, A2_PER)], rtmp2)

        def addrow(r, c2_):
            red2[pl.ds(r * 16, 16)] = red2[pl.ds(r * 16, 16)] + rtmp2[pl.ds(r * 16, 16)]
            return c2_

        lax.fori_loop(0, A2_PER // 16, addrow, 0, unroll=8)
        return carry

    lax.fori_loop(0, NSUB, red2_body, 0)

    def p7(r, carry):
        gx = red1[pl.ds(1024 + r * 16, 16)] + 0.05 * (red2[pl.ds(r * 16, 16)] - lux[pl.ds(r * 16, 16)])
        gy = red1[pl.ds(1280 + r * 16, 16)] + 0.05 * (red2[pl.ds(256 + r * 16, 16)] - luy[pl.ds(r * 16, 16)])
        gz = red1[pl.ds(1536 + r * 16, 16)] + 0.05 * (red2[pl.ds(512 + r * 16, 16)] - luz[pl.ds(r * 16, 16)])
        lg[pl.ds(r * 16, 16)] = gx
        lg[pl.ds(VPS + r * 16, 16)] = gy
        lg[pl.ds(2 * VPS + r * 16, 16)] = gz
        return carry

    lax.fori_loop(0, VPS // 16, p7, 0)

    @pl.when(cid == 0)
    def _():
        pltpu.sync_copy(lg.at[pl.ds(0, VPS)], out.at[pl.ds(s * VPS, VPS)])
        pltpu.sync_copy(lg.at[pl.ds(VPS, VPS)], out.at[pl.ds(N + s * VPS, VPS)])
        pltpu.sync_copy(lg.at[pl.ds(2 * VPS, VPS)], out.at[pl.ds(2 * N + s * VPS, VPS)])


def _build_mesh_sc(interpret=False):
    mesh = plsc.VectorSubcoreMesh(core_axis_name="c", subcore_axis_name="s",
                                  num_cores=2, num_subcores=NSUB)
    return pl.kernel(
        _mesh_sc_body,
        out_type=jax.ShapeDtypeStruct((3 * N,), jnp.float32),
        mesh=mesh,
        scratch_types=[
            pltpu.VMEM((N,), jnp.float32),
            pltpu.VMEM((N,), jnp.float32),
            pltpu.VMEM((N,), jnp.float32),
            pltpu.VMEM((FACES_PER_SUB,), jnp.int32),
            pltpu.VMEM((FACES_PER_SUB,), jnp.int32),
            pltpu.VMEM((FACES_PER_SUB,), jnp.int32),
            pltpu.VMEM((A1_TOT,), jnp.float32),
            pltpu.VMEM((A1_PER,), jnp.float32),
            pltpu.VMEM((A2_TOT,), jnp.float32),
            pltpu.VMEM((A2_PER,), jnp.float32),
            pltpu.VMEM((A1_PER,), jnp.float32),
            pltpu.VMEM((A2_PER,), jnp.float32),
            pltpu.VMEM((VPS,), jnp.float32),
            pltpu.VMEM((VPS,), jnp.float32),
            pltpu.VMEM((VPS,), jnp.float32),
            pltpu.VMEM((VPS,), jnp.float32),
            pltpu.VMEM((VPS,), jnp.float32),
            pltpu.VMEM((VPS,), jnp.float32),
            pltpu.VMEM((N,), jnp.float32),
            pltpu.VMEM((N,), jnp.float32),
            pltpu.VMEM((N,), jnp.float32),
            pltpu.VMEM((3 * VPS,), jnp.float32),
            pltpu.VMEM_SHARED((NSUB * A1_TOT,), jnp.float32),
            pltpu.VMEM_SHARED((3 * N,), jnp.float32),
            pltpu.VMEM_SHARED((NSUB * A2_TOT,), jnp.float32),
        ],
        compiler_params=pltpu.CompilerParams(
            use_tc_tiling_on_sc=False, needs_layout_passes=False),
        interpret=interpret,
    )


_mesh_sc = _build_mesh_sc()


def kernel(src_vts, src_faces, trg_vts, trg_color):
    del trg_color
    lr, mom = 0.01, 0.99
    faces_flat = src_faces.astype(jnp.int32).T.reshape(-1)
    pad = jnp.zeros((N, KPAD - 3), jnp.float32)
    trg_p = jnp.concatenate([trg_vts, pad], axis=1)
    trgT_p = trg_p.T
    t2 = jnp.sum(trg_vts * trg_vts, axis=1)[None, :]

    deform = jnp.zeros_like(src_vts)
    vel = jnp.zeros_like(src_vts)
    for _ in range(3):
        verts = src_vts + deform
        verts_p = jnp.concatenate([verts, pad], axis=1)
        g_ch = _chamfer_grad(verts_p, trgT_p, trg_p, t2)[:, :3]
        g_mesh = _mesh_sc(verts.T.reshape(-1), faces_flat).reshape(3, N).T
        g = 0.8 * g_ch + g_mesh
        vel = mom * vel + g
        deform = deform - lr * vel
    return src_vts + deform


# speedup vs baseline: 20.1482x; 1.1264x over previous
import functools

import jax
import jax.numpy as jnp
from jax import lax
from jax.experimental import pallas as pl
from jax.experimental.pallas import tpu as pltpu
from jax.experimental.pallas import tpu_sc as plsc

N = 4096
M = 4096
F = 16384
TILE = 256
NTILES = N // TILE
KPAD = 8

NSUB = 16
FACES_PER_SUB = F // NSUB
NGROUPS = FACES_PER_SUB // 16
VPS = N // NSUB
A1_PER = 7 * 256
A1_TOT = NSUB * A1_PER
A2_PER = 3 * 256
A2_TOT = NSUB * A2_PER


def _chamfer_grad_kernel(verts_ref, trgT_ref, trg_ref, t2_ref, out_ref, colmin_ref):
    phase = pl.program_id(0)
    tile = pl.program_id(1)
    vtile = verts_ref[...]
    g = jnp.dot(vtile, trgT_ref[...], preferred_element_type=jnp.float32)
    v2 = jnp.sum(vtile * vtile, axis=1, keepdims=True)
    d2 = v2 + t2_ref[...] - 2.0 * g

    @pl.when(phase == 0)
    def _():
        part = jnp.min(d2, axis=0, keepdims=True)

        @pl.when(tile == 0)
        def _():
            colmin_ref[...] = part

        @pl.when(tile > 0)
        def _():
            colmin_ref[...] = jnp.minimum(colmin_ref[...], part)

    @pl.when(phase == 1)
    def _():
        rowmin = jnp.min(d2, axis=1, keepdims=True)
        ohr = (d2 == rowmin).astype(jnp.float32)
        sel = jnp.dot(ohr, trg_ref[...], preferred_element_type=jnp.float32)
        ga = (2.0 / N) * (vtile - sel)
        c = (d2 == colmin_ref[...]).astype(jnp.float32)
        cnt = jnp.sum(c, axis=1, keepdims=True)
        selc = jnp.dot(c, trg_ref[...], preferred_element_type=jnp.float32)
        gb = (2.0 / M) * (vtile * cnt - selc)
        out_ref[...] = ga + gb


def _chamfer_grad(verts_p, trgT_p, trg_p, t2):
    return pl.pallas_call(
        _chamfer_grad_kernel,
        grid=(2, NTILES),
        in_specs=[
            pl.BlockSpec((TILE, KPAD), lambda p, t: (t, 0)),
            pl.BlockSpec((KPAD, M), lambda p, t: (0, 0)),
            pl.BlockSpec((M, KPAD), lambda p, t: (0, 0)),
            pl.BlockSpec((1, M), lambda p, t: (0, 0)),
        ],
        out_specs=pl.BlockSpec((TILE, KPAD), lambda p, t: (t, 0)),
        out_shape=jax.ShapeDtypeStruct((N, KPAD), jnp.float32),
        scratch_shapes=[pltpu.VMEM((1, M), jnp.float32)],
    )(verts_p, trgT_p, trg_p, t2)


def _sqrt16(n):
    i = plsc.bitcast(n, jnp.int32)
    y = plsc.bitcast(jnp.int32(0x5F3759DF) - lax.shift_right_logical(i, 1),
                     jnp.float32)
    half = 0.5 * n
    y = y * (1.5 - half * y * y)
    y = y * (1.5 - half * y * y)
    y = y * (1.5 - half * y * y)
    return n * y


def _zero_fill(ref, nrows):
    z = jnp.zeros((16,), jnp.float32)

    def body(r, carry):
        ref[pl.ds(r * 16, 16)] = z
        return carry

    lax.fori_loop(0, nrows, body, 0, unroll=8)


def _mesh_sc_body(vh, fh, out,
                  lv, lf0, lf1, lf2,
                  acc1, red1, red2, rtmp1,
                  lux, luy, luz, lwx, lwy, lwz,
                  fwx, fwy, fwz, lg,
                  shared1, sharedw):
    cid = lax.axis_index("c")
    s = lax.axis_index("s")

    pltpu.sync_copy(vh.at[pl.ds(0, N * KPAD)], lv)
    pltpu.sync_copy(fh.at[pl.ds(s * FACES_PER_SUB, FACES_PER_SUB)], lf0)
    pltpu.sync_copy(fh.at[pl.ds(F + s * FACES_PER_SUB, FACES_PER_SUB)], lf1)
    pltpu.sync_copy(fh.at[pl.ds(2 * F + s * FACES_PER_SUB, FACES_PER_SUB)], lf2)
    _zero_fill(acc1, A1_PER * NSUB // 16)

    two_over_f = jnp.float32(2.0 / F)

    def face1(gi, carry):
        base = gi * 16
        ia = lf0[pl.ds(base, 16)]
        ib = lf1[pl.ds(base, 16)]
        ic = lf2[pl.ds(base, 16)]
        ia8 = lax.shift_left(ia, 3)
        ib8 = lax.shift_left(ib, 3)
        ic8 = lax.shift_left(ic, 3)
        vax = plsc.load_gather(lv, [ia8])
        vay = plsc.load_gather(lv, [ia8 + 1])
        vaz = plsc.load_gather(lv, [ia8 + 2])
        vbx = plsc.load_gather(lv, [ib8])
        vby = plsc.load_gather(lv, [ib8 + 1])
        vbz = plsc.load_gather(lv, [ib8 + 2])
        vcx = plsc.load_gather(lv, [ic8])
        vcy = plsc.load_gather(lv, [ic8 + 1])
        vcz = plsc.load_gather(lv, [ic8 + 2])
        d0x, d0y, d0z = vax - vbx, vay - vby, vaz - vbz
        d1x, d1y, d1z = vbx - vcx, vby - vcy, vbz - vcz
        d2x, d2y, d2z = vcx - vax, vcy - vay, vcz - vaz
        e0 = _sqrt16(d0x * d0x + d0y * d0y + d0z * d0z)
        e1 = _sqrt16(d1x * d1x + d1y * d1y + d1z * d1z)
        e2 = _sqrt16(d2x * d2x + d2y * d2y + d2z * d2z)
        c0 = two_over_f * (2.0 * e0 - e1 - e2) / e0
        c1 = two_over_f * (2.0 * e1 - e0 - e2) / e1
        c2 = two_over_f * (2.0 * e2 - e0 - e1) / e2
        gv0x, gv0y, gv0z = c0 * d0x - c2 * d2x, c0 * d0y - c2 * d2y, c0 * d0z - c2 * d2z
        gv1x, gv1y, gv1z = c1 * d1x - c0 * d0x, c1 * d1y - c0 * d0y, c1 * d1z - c0 * d0z
        gv2x, gv2y, gv2z = c2 * d2x - c1 * d1x, c2 * d2y - c1 * d1y, c2 * d2z - c1 * d1z
        ja = lax.shift_right_logical(ia, 8) * A1_PER + jnp.bitwise_and(ia, 255)
        jb = lax.shift_right_logical(ib, 8) * A1_PER + jnp.bitwise_and(ib, 255)
        jc = lax.shift_right_logical(ic, 8) * A1_PER + jnp.bitwise_and(ic, 255)
        plsc.addupdate_scatter(acc1, [ja], vbx + vcx)
        plsc.addupdate_scatter(acc1, [ja + 256], vby + vcy)
        plsc.addupdate_scatter(acc1, [ja + 512], vbz + vcz)
        plsc.addupdate_scatter(acc1, [jb], vax + vcx)
        plsc.addupdate_scatter(acc1, [jb + 256], vay + vcy)
        plsc.addupdate_scatter(acc1, [jb + 512], vaz + vcz)
        plsc.addupdate_scatter(acc1, [jc], vax + vbx)
        plsc.addupdate_scatter(acc1, [jc + 256], vay + vby)
        plsc.addupdate_scatter(acc1, [jc + 512], vaz + vbz)
        twos = jnp.full((16,), 2.0, jnp.float32)
        plsc.addupdate_scatter(acc1, [ja + 768], twos)
        plsc.addupdate_scatter(acc1, [jb + 768], twos)
        plsc.addupdate_scatter(acc1, [jc + 768], twos)
        plsc.addupdate_scatter(acc1, [ja + 1024], gv0x)
        plsc.addupdate_scatter(acc1, [ja + 1280], gv0y)
        plsc.addupdate_scatter(acc1, [ja + 1536], gv0z)
        plsc.addupdate_scatter(acc1, [jb + 1024], gv1x)
        plsc.addupdate_scatter(acc1, [jb + 1280], gv1y)
        plsc.addupdate_scatter(acc1, [jb + 1536], gv1z)
        plsc.addupdate_scatter(acc1, [jc + 1024], gv2x)
        plsc.addupdate_scatter(acc1, [jc + 1280], gv2y)
        plsc.addupdate_scatter(acc1, [jc + 1536], gv2z)
        return carry

    lax.fori_loop(0, NGROUPS, face1, 0)

    pltpu.sync_copy(acc1, shared1.at[pl.ds(s * A1_TOT, A1_TOT)])
    plsc.subcore_barrier()
    _zero_fill(red1, A1_PER // 16)

    def red1_body(t, carry):
        pltpu.sync_copy(shared1.at[pl.ds(t * A1_TOT + s * A1_PER, A1_PER)], rtmp1)

        def addrow(r, c2_):
            red1[pl.ds(r * 16, 16)] = red1[pl.ds(r * 16, 16)] + rtmp1[pl.ds(r * 16, 16)]
            return c2_

        lax.fori_loop(0, A1_PER // 16, addrow, 0, unroll=8)
        return carry

    lax.fori_loop(0, NSUB, red1_body, 0)

    inv_n = jnp.float32(1.0 / N)

    def p3(r, carry):
        nbx = red1[pl.ds(r * 16, 16)]
        nby = red1[pl.ds(256 + r * 16, 16)]
        nbz = red1[pl.ds(512 + r * 16, 16)]
        dg = red1[pl.ds(768 + r * 16, 16)]
        degc = jnp.maximum(dg, 1.0)
        i8 = lax.shift_left(s * VPS + r * 16 + lax.iota(jnp.int32, 16), 3)
        vx = plsc.load_gather(lv, [i8])
        vy = plsc.load_gather(lv, [i8 + 1])
        vz = plsc.load_gather(lv, [i8 + 2])
        lapx = nbx / degc - vx
        lapy = nby / degc - vy
        lapz = nbz / degc - vz
        nl = _sqrt16(lapx * lapx + lapy * lapy + lapz * lapz)
        ux = inv_n * (lapx / nl)
        uy = inv_n * (lapy / nl)
        uz = inv_n * (lapz / nl)
        lux[pl.ds(r * 16, 16)] = ux
        luy[pl.ds(r * 16, 16)] = uy
        luz[pl.ds(r * 16, 16)] = uz
        lwx[pl.ds(r * 16, 16)] = ux / degc
        lwy[pl.ds(r * 16, 16)] = uy / degc
        lwz[pl.ds(r * 16, 16)] = uz / degc
        return carry

    lax.fori_loop(0, VPS // 16, p3, 0)

    pltpu.sync_copy(lwx, sharedw.at[pl.ds(s * VPS, VPS)])
    pltpu.sync_copy(lwy, sharedw.at[pl.ds(N + s * VPS, VPS)])
    pltpu.sync_copy(lwz, sharedw.at[pl.ds(2 * N + s * VPS, VPS)])
    plsc.subcore_barrier()
    pltpu.sync_copy(sharedw.at[pl.ds(0, N)], fwx)
    pltpu.sync_copy(sharedw.at[pl.ds(N, N)], fwy)
    pltpu.sync_copy(sharedw.at[pl.ds(2 * N, N)], fwz)
    _zero_fill(acc1, A2_PER * NSUB // 16)

    def face2(gi, carry):
        base = gi * 16
        ia = lf0[pl.ds(base, 16)]
        ib = lf1[pl.ds(base, 16)]
        ic = lf2[pl.ds(base, 16)]
        wax = plsc.load_gather(fwx, [ia])
        way = plsc.load_gather(fwy, [ia])
        waz = plsc.load_gather(fwz, [ia])
        wbx = plsc.load_gather(fwx, [ib])
        wby = plsc.load_gather(fwy, [ib])
        wbz = plsc.load_gather(fwz, [ib])
        wcx = plsc.load_gather(fwx, [ic])
        wcy = plsc.load_gather(fwy, [ic])
        wcz = plsc.load_gather(fwz, [ic])
        ja = lax.shift_right_logical(ia, 8) * A2_PER + jnp.bitwise_and(ia, 255)
        jb = lax.shift_right_logical(ib, 8) * A2_PER + jnp.bitwise_and(ib, 255)
        jc = lax.shift_right_logical(ic, 8) * A2_PER + jnp.bitwise_and(ic, 255)
        plsc.addupdate_scatter(acc1, [ja], wbx + wcx)
        plsc.addupdate_scatter(acc1, [ja + 256], wby + wcy)
        plsc.addupdate_scatter(acc1, [ja + 512], wbz + wcz)
        plsc.addupdate_scatter(acc1, [jb], wax + wcx)
        plsc.addupdate_scatter(acc1, [jb + 256], way + wcy)
        plsc.addupdate_scatter(acc1, [jb + 512], waz + wcz)
        plsc.addupdate_scatter(acc1, [jc], wax + wbx)
        plsc.addupdate_scatter(acc1, [jc + 256], way + wby)
        plsc.addupdate_scatter(acc1, [jc + 512], waz + wbz)
        return carry

    lax.fori_loop(0, NGROUPS, face2, 0)

    pltpu.sync_copy(acc1.at[pl.ds(0, A2_TOT)], shared1.at[pl.ds(s * A2_TOT, A2_TOT)])
    plsc.subcore_barrier()
    _zero_fill(red2, A2_PER // 16)

    def red2_body(t, carry):
        pltpu.sync_copy(shared1.at[pl.ds(t * A2_TOT + s * A2_PER, A2_PER)],
                        rtmp1.at[pl.ds(0, A2_PER)])

        def addrow(r, c2_):
            red2[pl.ds(r * 16, 16)] = red2[pl.ds(r * 16, 16)] + rtmp1[pl.ds(r * 16, 16)]
            return c2_

        lax.fori_loop(0, A2_PER // 16, addrow, 0, unroll=8)
        return carry

    lax.fori_loop(0, NSUB, red2_body, 0)

    _zero_fill(lg, VPS * KPAD // 16)

    def p7(r, carry):
        gx = red1[pl.ds(1024 + r * 16, 16)] + 0.05 * (red2[pl.ds(r * 16, 16)] - lux[pl.ds(r * 16, 16)])
        gy = red1[pl.ds(1280 + r * 16, 16)] + 0.05 * (red2[pl.ds(256 + r * 16, 16)] - luy[pl.ds(r * 16, 16)])
        gz = red1[pl.ds(1536 + r * 16, 16)] + 0.05 * (red2[pl.ds(512 + r * 16, 16)] - luz[pl.ds(r * 16, 16)])
        o8 = lax.shift_left(r * 16 + lax.iota(jnp.int32, 16), 3)
        plsc.store_scatter(lg, [o8], gx)
        plsc.store_scatter(lg, [o8 + 1], gy)
        plsc.store_scatter(lg, [o8 + 2], gz)
        return carry

    lax.fori_loop(0, VPS // 16, p7, 0)

    @pl.when(cid == 0)
    def _():
        pltpu.sync_copy(lg, out.at[pl.ds(s * VPS * KPAD, VPS * KPAD)])


def _build_mesh_sc(interpret=False):
    mesh = plsc.VectorSubcoreMesh(core_axis_name="c", subcore_axis_name="s",
                                  num_cores=2, num_subcores=NSUB)
    return pl.kernel(
        _mesh_sc_body,
        out_type=jax.ShapeDtypeStruct((N * KPAD,), jnp.float32),
        mesh=mesh,
        scratch_types=[
            pltpu.VMEM((N * KPAD,), jnp.float32),
            pltpu.VMEM((FACES_PER_SUB,), jnp.int32),
            pltpu.VMEM((FACES_PER_SUB,), jnp.int32),
            pltpu.VMEM((FACES_PER_SUB,), jnp.int32),
            pltpu.VMEM((A1_TOT,), jnp.float32),
            pltpu.VMEM((A1_PER,), jnp.float32),
            pltpu.VMEM((A2_PER,), jnp.float32),
            pltpu.VMEM((A1_PER,), jnp.float32),
            pltpu.VMEM((VPS,), jnp.float32),
            pltpu.VMEM((VPS,), jnp.float32),
            pltpu.VMEM((VPS,), jnp.float32),
            pltpu.VMEM((VPS,), jnp.float32),
            pltpu.VMEM((VPS,), jnp.float32),
            pltpu.VMEM((VPS,), jnp.float32),
            pltpu.VMEM((N,), jnp.float32),
            pltpu.VMEM((N,), jnp.float32),
            pltpu.VMEM((N,), jnp.float32),
            pltpu.VMEM((VPS * KPAD,), jnp.float32),
            pltpu.VMEM_SHARED((NSUB * A1_TOT,), jnp.float32),
            pltpu.VMEM_SHARED((3 * N,), jnp.float32),
        ],
        compiler_params=pltpu.CompilerParams(
            use_tc_tiling_on_sc=False, needs_layout_passes=False),
        interpret=interpret,
    )


_mesh_sc = _build_mesh_sc()


def kernel(src_vts, src_faces, trg_vts, trg_color):
    del trg_color
    lr, mom = 0.01, 0.99
    faces_flat = src_faces.astype(jnp.int32).T.reshape(-1)
    pad = jnp.zeros((N, KPAD - 3), jnp.float32)
    src_p = jnp.concatenate([src_vts, pad], axis=1)
    trg_p = jnp.concatenate([trg_vts, pad], axis=1)
    trgT_p = trg_p.T
    t2 = jnp.sum(trg_vts * trg_vts, axis=1)[None, :]

    deform = jnp.zeros((N, KPAD), jnp.float32)
    vel = jnp.zeros((N, KPAD), jnp.float32)
    for _ in range(3):
        verts_p = src_p + deform
        g_ch = _chamfer_grad(verts_p, trgT_p, trg_p, t2)
        g_mesh = _mesh_sc(verts_p.reshape(-1), faces_flat).reshape(N, KPAD)
        g = 0.8 * g_ch + g_mesh
        vel = mom * vel + g
        deform = deform - lr * vel
    return (src_p + deform)[:, :3]
